# baseline (speedup 1.0000x reference)
import functools

import jax
import jax.numpy as jnp
from jax import lax
from jax.experimental import pallas as pl
from jax.experimental.pallas import tpu as pltpu
from jax.experimental.pallas import tpu_sc as plsc

NUM_EMB = 1000000
DIM = 16
NC = 2
NS = 16
NW = NC * NS


def _sc_gather_sample(mu, log_sigma, eps, idx_flat):
    n = idx_flat.shape[0]
    assert n % NW == 0
    bpw = n // NW
    ch = 1024
    assert bpw % ch == 0
    n_chunks = bpw // ch

    mesh = plsc.VectorSubcoreMesh(
        core_axis_name="c", subcore_axis_name="s",
        num_cores=NC, num_subcores=NS)

    @functools.partial(
        pl.kernel,
        mesh=mesh,
        compiler_params=pltpu.CompilerParams(use_tc_tiling_on_sc=False),
        out_type=jax.ShapeDtypeStruct((n, DIM), jnp.float32),
        scratch_types=[
            pltpu.VMEM((bpw,), jnp.int32),
            pltpu.VMEM((ch, DIM), jnp.float32),
            pltpu.VMEM((ch, DIM), jnp.float32),
            pltpu.VMEM((ch, DIM), jnp.float32),
            pltpu.SemaphoreType.DMA,
        ],
    )
    def k(mu_hbm, ls_hbm, eps_hbm, idx_hbm, out_hbm,
          idx_v, mu_v, ls_v, eps_v, sem):
        wid = lax.axis_index("s") * NC + lax.axis_index("c")
        base = wid * bpw
        pltpu.sync_copy(idx_hbm.at[pl.ds(base, bpw)], idx_v)

        def chunk(c, carry):
            off = c * ch
            idx_chunk = idx_v.at[pl.ds(off, ch)]
            cp1 = pltpu.async_copy(mu_hbm.at[idx_chunk], mu_v, sem)
            cp2 = pltpu.async_copy(ls_hbm.at[idx_chunk], ls_v, sem)
            cp3 = pltpu.async_copy(eps_hbm.at[idx_chunk], eps_v, sem)
            cp1.wait()
            cp2.wait()
            cp3.wait()

            def row(i, carry2):
                mu_v[i, :] = mu_v[i, :] + jnp.exp(ls_v[i, :]) * eps_v[i, :]
                return carry2

            lax.fori_loop(0, ch, row, 0, unroll=4)
            pltpu.sync_copy(mu_v, out_hbm.at[pl.ds(base + off, ch)])
            return carry

        lax.fori_loop(0, n_chunks, chunk, 0)

    return k(mu, log_sigma, eps, idx_flat)


def _tc_kl(mu, log_sigma):
    rows, cols = 1000, 16000
    br = 40
    grid = rows // br
    mu2 = mu.reshape(rows, cols)
    ls2 = log_sigma.reshape(rows, cols)

    def body(mu_ref, ls_ref, acc_ref):
        @pl.when(pl.program_id(0) == 0)
        def _():
            acc_ref[...] = jnp.zeros((1, 1), jnp.float32)

        m = mu_ref[...]
        l = ls_ref[...]
        term = jnp.exp(2.0 * l) + m * m - 1.0 - 2.0 * l
        acc_ref[...] += jnp.sum(term).reshape(1, 1)

    out = pl.pallas_call(
        body,
        grid=(grid,),
        in_specs=[
            pl.BlockSpec((br, cols), lambda i: (i, 0)),
            pl.BlockSpec((br, cols), lambda i: (i, 0)),
        ],
        out_specs=pl.BlockSpec((1, 1), lambda i: (0, 0)),
        out_shape=jax.ShapeDtypeStruct((1, 1), jnp.float32),
    )(mu2, ls2)
    return 0.5 * out[0, 0]


def kernel(input, mu, log_sigma, eps):
    idx_flat = input.reshape(-1)
    emb_flat = _sc_gather_sample(mu, log_sigma, eps, idx_flat)
    embedding = emb_flat.reshape(input.shape + (DIM,))
    kl = _tc_kl(mu, log_sigma)
    return (embedding, kl)


# speedup vs baseline: 2.1846x; 2.1846x over previous
import functools

import jax
import jax.numpy as jnp
from jax import lax
from jax.experimental import pallas as pl
from jax.experimental.pallas import tpu as pltpu
from jax.experimental.pallas import tpu_sc as plsc

NUM_EMB = 1000000
DIM = 16
NC = 2
NS = 16
NW = NC * NS


def _tc_sample_kl(mu_t, ls_t, eps_t):
    bc = 16384
    grid = (NUM_EMB + bc - 1) // bc

    def body(mu_ref, ls_ref, eps_ref, w_ref, acc_ref):
        i = pl.program_id(0)
        m = mu_ref[...]
        l = ls_ref[...]
        e = eps_ref[...]
        sig = jnp.exp(l)
        w_ref[...] = (m + sig * e).T
        col = i * bc + lax.broadcasted_iota(jnp.int32, (DIM, bc), 1)
        term = jnp.where(col < NUM_EMB,
                         sig * sig + m * m - 1.0 - 2.0 * l, 0.0)

        @pl.when(i == 0)
        def _():
            acc_ref[...] = jnp.zeros((1, 1), jnp.float32)

        acc_ref[...] += jnp.sum(term).reshape(1, 1)

    return pl.pallas_call(
        body,
        grid=(grid,),
        in_specs=[
            pl.BlockSpec((DIM, bc), lambda i: (0, i)),
            pl.BlockSpec((DIM, bc), lambda i: (0, i)),
            pl.BlockSpec((DIM, bc), lambda i: (0, i)),
        ],
        out_specs=[
            pl.BlockSpec((bc, DIM), lambda i: (i, 0)),
            pl.BlockSpec((1, 1), lambda i: (0, 0)),
        ],
        out_shape=[
            jax.ShapeDtypeStruct((NUM_EMB, DIM), jnp.float32),
            jax.ShapeDtypeStruct((1, 1), jnp.float32),
        ],
    )(mu_t, ls_t, eps_t)


def _sc_gather_rows(w, idx_flat):
    n = idx_flat.shape[0]
    bpw = n // NW
    ch = 1024
    n_chunks = bpw // ch

    mesh = plsc.VectorSubcoreMesh(
        core_axis_name="c", subcore_axis_name="s",
        num_cores=NC, num_subcores=NS)

    @functools.partial(
        pl.kernel,
        mesh=mesh,
        compiler_params=pltpu.CompilerParams(use_tc_tiling_on_sc=False),
        out_type=jax.ShapeDtypeStruct((n, DIM), jnp.float32),
        scratch_types=[
            pltpu.VMEM((bpw,), jnp.int32),
            pltpu.VMEM((ch, DIM), jnp.float32),
            pltpu.VMEM((ch, DIM), jnp.float32),
            pltpu.SemaphoreType.DMA,
            pltpu.SemaphoreType.DMA,
        ],
    )
    def k(w_hbm, idx_hbm, out_hbm, idx_v, buf0, buf1, sem0, sem1):
        wid = lax.axis_index("s") * NC + lax.axis_index("c")
        base = wid * bpw
        pltpu.sync_copy(idx_hbm.at[pl.ds(base, bpw)], idx_v)
        bufs = (buf0, buf1)
        sems = (sem0, sem1)
        cps = [None, None]
        cps[0] = pltpu.async_copy(w_hbm.at[idx_v.at[pl.ds(0, ch)]],
                                  buf0, sem0)
        for g in range(n_chunks):
            if g + 1 < n_chunks:
                cps[(g + 1) % 2] = pltpu.async_copy(
                    w_hbm.at[idx_v.at[pl.ds((g + 1) * ch, ch)]],
                    bufs[(g + 1) % 2], sems[(g + 1) % 2])
            cps[g % 2].wait()
            pltpu.sync_copy(bufs[g % 2], out_hbm.at[pl.ds(base + g * ch, ch)])

    return k(w, idx_flat)


def kernel(input, mu, log_sigma, eps):
    w, kl_acc = _tc_sample_kl(mu.T, log_sigma.T, eps.T)
    idx_flat = input.reshape(-1)
    rows = _sc_gather_rows(w, idx_flat)
    embedding = rows.reshape(input.shape + (DIM,))
    kl = 0.5 * kl_acc[0, 0]
    return (embedding, kl)
